# baseline (speedup 1.0000x reference)
import functools

import numpy as np

import jax
import jax.numpy as jnp
from jax import lax
from jax.experimental import pallas as pl
from jax.experimental.pallas import tpu as pltpu
from jax.experimental.pallas import tpu_sc as plsc

M = 2048
N = 4096
NV = N // 16
NB = 256
NC = 2
NS = 16
NW = NC * NS
RPW = M // NW
MINI32 = np.int32(-2147483648)


def _bitcast(v, dt):
    return lax.bitcast_convert_type(v, dt)


def _to_key(v):
    xi = _bitcast(v, jnp.int32)
    mask = (xi >> 31) | MINI32
    return xi ^ mask


def _from_key(k):
    mask = ((~k) >> 31) | MINI32
    return _bitcast(k ^ mask, jnp.float32)


def _digit(k, shift):
    ku = _bitcast(k, jnp.uint32)
    return ((ku >> shift) & 255).astype(jnp.int32)


def _lane15(v):
    lane = lax.iota(jnp.int32, 16)
    return jnp.sum(jnp.where(lane == 15, v, jnp.zeros((16,), v.dtype)))


def _store_scalar(ref, idx, val, lane):
    idxv = jnp.broadcast_to(idx, (16,)).astype(jnp.int32)
    valv = jnp.broadcast_to(val, (16,))
    plsc.store_scatter(ref, [idxv], valv, mask=lane == 0)


def _load_scalar_bcast(ref, idx):
    idxv = jnp.broadcast_to(idx, (16,)).astype(jnp.int32)
    return plsc.load_gather(ref, [idxv])


def _sc_body(x_hbm, a_hbm, out_hbm,
             xin, ain, kx0, kx1, ka0, ka1,
             cntx, cnta, totx, tota, resm, resw, resd):
    wid = lax.axis_index("s") * NC + lax.axis_index("c")
    base = wid * RPW
    lane = lax.iota(jnp.int32, 16)
    stride_idx = lane * NV
    zeros16 = jnp.zeros((16,), jnp.int32)
    ones16 = jnp.ones((16,), jnp.int32)

    def radix_pass(shift, sx_ref, dx_ref, sa_ref, da_ref):
        def zero(i, c):
            cntx[pl.ds(i * 16, 16)] = zeros16
            cnta[pl.ds(i * 16, 16)] = zeros16
            return c
        lax.fori_loop(0, NB, zero, 0)

        def hist(i, c):
            kxv = plsc.load_gather(sx_ref, [stride_idx + i])
            kav = plsc.load_gather(sa_ref, [stride_idx + i])
            dx = _digit(kxv, shift)
            da = _digit(kav, shift)
            plsc.addupdate_scatter(cntx, [dx * 16 + lane], ones16)
            plsc.addupdate_scatter(cnta, [da * 16 + lane], ones16)
            return c
        lax.fori_loop(0, NV, hist, 0)

        def tot(i, c):
            _store_scalar(totx, i, jnp.sum(cntx[pl.ds(i * 16, 16)]), lane)
            _store_scalar(tota, i, jnp.sum(cnta[pl.ds(i * 16, 16)]), lane)
            return c
        lax.fori_loop(0, NB, tot, 0)

        def scan(j, carry):
            cx, ca = carry
            vx = totx[pl.ds(j * 16, 16)]
            va = tota[pl.ds(j * 16, 16)]
            totx[pl.ds(j * 16, 16)] = plsc.cumsum(vx) - vx + cx
            tota[pl.ds(j * 16, 16)] = plsc.cumsum(va) - va + ca
            return cx + jnp.sum(vx), ca + jnp.sum(va)
        lax.fori_loop(0, NB // 16, scan,
                      (jnp.int32(0), jnp.int32(0)))

        def offs(i, c):
            vx = cntx[pl.ds(i * 16, 16)]
            va = cnta[pl.ds(i * 16, 16)]
            bx = _load_scalar_bcast(totx, i)
            ba = _load_scalar_bcast(tota, i)
            cntx[pl.ds(i * 16, 16)] = plsc.cumsum(vx) - vx + bx
            cnta[pl.ds(i * 16, 16)] = plsc.cumsum(va) - va + ba
            return c
        lax.fori_loop(0, NB, offs, 0)

        def perm(i, c):
            kxv = plsc.load_gather(sx_ref, [stride_idx + i])
            kav = plsc.load_gather(sa_ref, [stride_idx + i])
            dx = _digit(kxv, shift)
            da = _digit(kav, shift)
            cix = dx * 16 + lane
            cia = da * 16 + lane
            destx = plsc.load_gather(cntx, [cix])
            desta = plsc.load_gather(cnta, [cia])
            plsc.store_scatter(cntx, [cix], destx + 1)
            plsc.store_scatter(cnta, [cia], desta + 1)
            plsc.store_scatter(dx_ref, [destx], kxv)
            plsc.store_scatter(da_ref, [desta], kav)
            return c
        lax.fori_loop(0, NV, perm, 0)

    def row_body(r, c):
        row = base + r
        pltpu.sync_copy(x_hbm.at[row], xin)
        pltpu.sync_copy(a_hbm.at[row], ain)

        def conv(i, carry):
            sx, sa = carry
            vx = xin[pl.ds(i * 16, 16)]
            va = ain[pl.ds(i * 16, 16)]
            kx0[pl.ds(i * 16, 16)] = _to_key(vx)
            ka0[pl.ds(i * 16, 16)] = _to_key(va)
            return sx + vx, sa + va
        sumx, suma = lax.fori_loop(
            0, NV, conv,
            (jnp.zeros((16,), jnp.float32), jnp.zeros((16,), jnp.float32)))

        radix_pass(0, kx0, kx1, ka0, ka1)
        radix_pass(8, kx1, kx0, ka1, ka0)
        radix_pass(16, kx0, kx1, ka0, ka1)
        radix_pass(24, kx1, kx0, ka1, ka0)

        def wacc(i, acc):
            fx = _from_key(kx0[pl.ds(i * 16, 16)])
            fa = _from_key(ka0[pl.ds(i * 16, 16)])
            return acc + jnp.abs(fx - fa)
        acc = lax.fori_loop(0, NV, wacc, jnp.zeros((16,), jnp.float32))

        med_off = ((N - 1) // 2 // 16) * 16
        medx = _lane15(_from_key(kx0[pl.ds(med_off, 16)]))
        meda = _lane15(_from_key(ka0[pl.ds(med_off, 16)]))
        med_d = medx - meda
        sgn = jnp.sign(med_d)
        inv_n = np.float32(1.0 / N)
        _store_scalar(resm, r, (jnp.sum(sumx) - jnp.sum(suma)) * inv_n * sgn, lane)
        _store_scalar(resw, r, jnp.sum(acc) * inv_n * sgn, lane)
        _store_scalar(resd, r, med_d, lane)
        return c

    lax.fori_loop(0, RPW, row_body, 0)

    pltpu.sync_copy(resm, out_hbm.at[0, pl.ds(base, RPW)])
    pltpu.sync_copy(resw, out_hbm.at[1, pl.ds(base, RPW)])
    pltpu.sync_copy(resd, out_hbm.at[2, pl.ds(base, RPW)])


@functools.lru_cache(maxsize=None)
def _build():
    return pl.kernel(
        _sc_body,
        out_type=jax.ShapeDtypeStruct((3, M), jnp.float32),
        mesh=plsc.VectorSubcoreMesh(core_axis_name="c", subcore_axis_name="s"),
        compiler_params=pltpu.CompilerParams(needs_layout_passes=False),
        scratch_types=[
            pltpu.VMEM((N,), jnp.float32),
            pltpu.VMEM((N,), jnp.float32),
            pltpu.VMEM((N,), jnp.int32),
            pltpu.VMEM((N,), jnp.int32),
            pltpu.VMEM((N,), jnp.int32),
            pltpu.VMEM((N,), jnp.int32),
            pltpu.VMEM((NB * 16,), jnp.int32),
            pltpu.VMEM((NB * 16,), jnp.int32),
            pltpu.VMEM((NB,), jnp.int32),
            pltpu.VMEM((NB,), jnp.int32),
            pltpu.VMEM((RPW,), jnp.float32),
            pltpu.VMEM((RPW,), jnp.float32),
            pltpu.VMEM((RPW,), jnp.float32),
        ],
    )


def kernel(x, anchor_features):
    return _build()(x, anchor_features)


# speedup vs baseline: 1.2833x; 1.2833x over previous
import functools

import numpy as np

import jax
import jax.numpy as jnp
from jax import lax
from jax.experimental import pallas as pl
from jax.experimental.pallas import tpu as pltpu
from jax.experimental.pallas import tpu_sc as plsc

M = 2048
N = 4096
NV = N // 16
NB = 256
NC = 2
NS = 16
NW = NC * NS
RPW = M // NW
MINI32 = np.int32(-2147483648)


def _bitcast(v, dt):
    return lax.bitcast_convert_type(v, dt)


def _to_key(v):
    xi = _bitcast(v, jnp.int32)
    mask = (xi >> 31) | MINI32
    return xi ^ mask


def _from_key(k):
    mask = ((~k) >> 31) | MINI32
    return _bitcast(k ^ mask, jnp.float32)


def _digit(k, shift):
    ku = _bitcast(k, jnp.uint32)
    return ((ku >> shift) & 255).astype(jnp.int32)


def _lane15(v):
    lane = lax.iota(jnp.int32, 16)
    return jnp.sum(jnp.where(lane == 15, v, jnp.zeros((16,), v.dtype)))


def _store_scalar(ref, idx, val, lane):
    idxv = jnp.broadcast_to(idx, (16,)).astype(jnp.int32)
    valv = jnp.broadcast_to(val, (16,))
    plsc.store_scatter(ref, [idxv], valv, mask=lane == 0)


def _load_scalar_bcast(ref, idx):
    idxv = jnp.broadcast_to(idx, (16,)).astype(jnp.int32)
    return plsc.load_gather(ref, [idxv])


def _sc_body(x_hbm, a_hbm, out_hbm,
             xin, ain, kx0, kx1, ka0, ka1,
             cntx, cnta, resm, resw, resd):
    wid = lax.axis_index("s") * NC + lax.axis_index("c")
    base = wid * RPW
    lane = lax.iota(jnp.int32, 16)
    stride_idx = lane * NV
    zeros16 = jnp.zeros((16,), jnp.int32)
    ones16 = jnp.ones((16,), jnp.int32)

    def radix_pass(shift, sx_ref, dx_ref, sa_ref, da_ref):
        def zero(i, c):
            cntx[pl.ds(i * 16, 16)] = zeros16
            cnta[pl.ds(i * 16, 16)] = zeros16
            return c
        lax.fori_loop(0, NB, zero, 0, unroll=8)

        def hist(i, c):
            kxv = plsc.load_gather(sx_ref, [stride_idx + i])
            kav = plsc.load_gather(sa_ref, [stride_idx + i])
            dx = _digit(kxv, shift)
            da = _digit(kav, shift)
            plsc.addupdate_scatter(cntx, [dx * 16 + lane], ones16)
            plsc.addupdate_scatter(cnta, [da * 16 + lane], ones16)
            return c
        lax.fori_loop(0, NV, hist, 0, unroll=8)

        def prefix(i, carry):
            cx, ca = carry
            vx = cntx[pl.ds(i * 16, 16)]
            va = cnta[pl.ds(i * 16, 16)]
            cntx[pl.ds(i * 16, 16)] = plsc.cumsum(vx) - vx + cx
            cnta[pl.ds(i * 16, 16)] = plsc.cumsum(va) - va + ca
            return cx + jnp.sum(vx), ca + jnp.sum(va)
        lax.fori_loop(0, NB, prefix, (jnp.int32(0), jnp.int32(0)), unroll=8)

        def perm(i, c):
            kxv = plsc.load_gather(sx_ref, [stride_idx + i])
            kav = plsc.load_gather(sa_ref, [stride_idx + i])
            dx = _digit(kxv, shift)
            da = _digit(kav, shift)
            cix = dx * 16 + lane
            cia = da * 16 + lane
            destx = plsc.load_gather(cntx, [cix])
            desta = plsc.load_gather(cnta, [cia])
            plsc.store_scatter(cntx, [cix], destx + 1)
            plsc.store_scatter(cnta, [cia], desta + 1)
            plsc.store_scatter(dx_ref, [destx], kxv)
            plsc.store_scatter(da_ref, [desta], kav)
            return c
        lax.fori_loop(0, NV, perm, 0, unroll=4)

    def row_body(r, c):
        row = base + r
        pltpu.sync_copy(x_hbm.at[row], xin)
        pltpu.sync_copy(a_hbm.at[row], ain)

        def conv(i, carry):
            sx, sa = carry
            vx = xin[pl.ds(i * 16, 16)]
            va = ain[pl.ds(i * 16, 16)]
            kx0[pl.ds(i * 16, 16)] = _to_key(vx)
            ka0[pl.ds(i * 16, 16)] = _to_key(va)
            return sx + vx, sa + va
        sumx, suma = lax.fori_loop(
            0, NV, conv,
            (jnp.zeros((16,), jnp.float32), jnp.zeros((16,), jnp.float32)),
            unroll=8)

        radix_pass(0, kx0, kx1, ka0, ka1)
        radix_pass(8, kx1, kx0, ka1, ka0)
        radix_pass(16, kx0, kx1, ka0, ka1)
        radix_pass(24, kx1, kx0, ka1, ka0)

        def wacc(i, acc):
            fx = _from_key(kx0[pl.ds(i * 16, 16)])
            fa = _from_key(ka0[pl.ds(i * 16, 16)])
            return acc + jnp.abs(fx - fa)
        acc = lax.fori_loop(0, NV, wacc, jnp.zeros((16,), jnp.float32),
                            unroll=8)

        med_off = ((N - 1) // 2 // 16) * 16
        medx = _lane15(_from_key(kx0[pl.ds(med_off, 16)]))
        meda = _lane15(_from_key(ka0[pl.ds(med_off, 16)]))
        med_d = medx - meda
        sgn = jnp.sign(med_d)
        inv_n = np.float32(1.0 / N)
        _store_scalar(resm, r, (jnp.sum(sumx) - jnp.sum(suma)) * inv_n * sgn, lane)
        _store_scalar(resw, r, jnp.sum(acc) * inv_n * sgn, lane)
        _store_scalar(resd, r, med_d, lane)
        return c

    lax.fori_loop(0, RPW, row_body, 0)

    pltpu.sync_copy(resm, out_hbm.at[0, pl.ds(base, RPW)])
    pltpu.sync_copy(resw, out_hbm.at[1, pl.ds(base, RPW)])
    pltpu.sync_copy(resd, out_hbm.at[2, pl.ds(base, RPW)])


@functools.lru_cache(maxsize=None)
def _build():
    return pl.kernel(
        _sc_body,
        out_type=jax.ShapeDtypeStruct((3, M), jnp.float32),
        mesh=plsc.VectorSubcoreMesh(core_axis_name="c", subcore_axis_name="s"),
        compiler_params=pltpu.CompilerParams(needs_layout_passes=False),
        scratch_types=[
            pltpu.VMEM((N,), jnp.float32),
            pltpu.VMEM((N,), jnp.float32),
            pltpu.VMEM((N,), jnp.int32),
            pltpu.VMEM((N,), jnp.int32),
            pltpu.VMEM((N,), jnp.int32),
            pltpu.VMEM((N,), jnp.int32),
            pltpu.VMEM((NB * 16,), jnp.int32),
            pltpu.VMEM((NB * 16,), jnp.int32),
            pltpu.VMEM((RPW,), jnp.float32),
            pltpu.VMEM((RPW,), jnp.float32),
            pltpu.VMEM((RPW,), jnp.float32),
        ],
    )


def kernel(x, anchor_features):
    return _build()(x, anchor_features)


# speedup vs baseline: 2.4459x; 1.9059x over previous
import functools

import numpy as np

import jax
import jax.numpy as jnp
from jax import lax
from jax.experimental import pallas as pl
from jax.experimental.pallas import tpu as pltpu
from jax.experimental.pallas import tpu_sc as plsc

M = 2048
N = 4096
NV = N // 16
NB = 256
NC = 2
NS = 16
NW = NC * NS
RPW = M // NW
MINI32 = np.int32(-2147483648)


def _bitcast(v, dt):
    return lax.bitcast_convert_type(v, dt)


def _to_key(v):
    xi = _bitcast(v, jnp.int32)
    mask = (xi >> 31) | MINI32
    return xi ^ mask


def _from_key(k):
    mask = ((~k) >> 31) | MINI32
    return _bitcast(k ^ mask, jnp.float32)


def _digit(k, shift):
    ku = _bitcast(k, jnp.uint32)
    return ((ku >> shift) & 255).astype(jnp.int32)


def _lane15(v):
    lane = lax.iota(jnp.int32, 16)
    return jnp.sum(jnp.where(lane == 15, v, jnp.zeros((16,), v.dtype)))


def _store_scalar(ref, idx, val, lane):
    idxv = jnp.broadcast_to(idx, (16,)).astype(jnp.int32)
    valv = jnp.broadcast_to(val, (16,))
    plsc.store_scatter(ref, [idxv], valv, mask=lane == 0)


def _load_scalar_bcast(ref, idx):
    idxv = jnp.broadcast_to(idx, (16,)).astype(jnp.int32)
    return plsc.load_gather(ref, [idxv])


def _sc_body(x_hbm, a_hbm, out_hbm,
             xin, ain, kx0, kx1, ka0, ka1,
             cntx, cnta, resm, resw, resd):
    wid = lax.axis_index("s") * NC + lax.axis_index("c")
    base = wid * RPW
    lane = lax.iota(jnp.int32, 16)
    stride_idx = lane * (NV + 1)
    zeros16 = jnp.zeros((16,), jnp.int32)
    ones16 = jnp.ones((16,), jnp.int32)

    def radix_pass(shift, sx_ref, dx_ref, sa_ref, da_ref):
        def zero(i, c):
            cntx[pl.ds(i * 16, 16)] = zeros16
            cnta[pl.ds(i * 16, 16)] = zeros16
            return c
        lax.fori_loop(0, NB, zero, 0, unroll=8)

        def hist(i, c):
            kxv = plsc.load_gather(sx_ref, [stride_idx + i])
            kav = plsc.load_gather(sa_ref, [stride_idx + i])
            dx = _digit(kxv, shift)
            da = _digit(kav, shift)
            plsc.addupdate_scatter(cntx, [dx * 16 + lane], ones16)
            plsc.addupdate_scatter(cnta, [da * 16 + lane], ones16)
            return c
        lax.fori_loop(0, NV, hist, 0, unroll=8)

        def prefix(i, carry):
            cx, ca = carry
            vx = cntx[pl.ds(i * 16, 16)]
            va = cnta[pl.ds(i * 16, 16)]
            cntx[pl.ds(i * 16, 16)] = plsc.cumsum(vx) - vx + cx
            cnta[pl.ds(i * 16, 16)] = plsc.cumsum(va) - va + ca
            return cx + jnp.sum(vx), ca + jnp.sum(va)
        lax.fori_loop(0, NB, prefix, (jnp.int32(0), jnp.int32(0)), unroll=8)

        def perm(i, c):
            kxv = plsc.load_gather(sx_ref, [stride_idx + i])
            kav = plsc.load_gather(sa_ref, [stride_idx + i])
            dx = _digit(kxv, shift)
            da = _digit(kav, shift)
            cix = dx * 16 + lane
            cia = da * 16 + lane
            destx = plsc.load_gather(cntx, [cix])
            desta = plsc.load_gather(cnta, [cia])
            plsc.store_scatter(cntx, [cix], destx + 1)
            plsc.store_scatter(cnta, [cia], desta + 1)
            plsc.store_scatter(dx_ref, [destx + (destx >> 8)], kxv)
            plsc.store_scatter(da_ref, [desta + (desta >> 8)], kav)
            return c
        lax.fori_loop(0, NV, perm, 0, unroll=4)

    def row_body(r, c):
        row = base + r
        pltpu.sync_copy(x_hbm.at[row], xin)
        pltpu.sync_copy(a_hbm.at[row], ain)

        def conv(i, carry):
            sx, sa = carry
            vx = xin[pl.ds(i * 16, 16)]
            va = ain[pl.ds(i * 16, 16)]
            qb = i * 16 + (i >> 4)
            kx0[pl.ds(qb, 16)] = _to_key(vx)
            ka0[pl.ds(qb, 16)] = _to_key(va)
            return sx + vx, sa + va
        sumx, suma = lax.fori_loop(
            0, NV, conv,
            (jnp.zeros((16,), jnp.float32), jnp.zeros((16,), jnp.float32)),
            unroll=8)

        radix_pass(0, kx0, kx1, ka0, ka1)
        radix_pass(8, kx1, kx0, ka1, ka0)
        radix_pass(16, kx0, kx1, ka0, ka1)
        radix_pass(24, kx1, kx0, ka1, ka0)

        def wacc(i, acc):
            qb = i * 16 + (i >> 4)
            fx = _from_key(kx0[pl.ds(qb, 16)])
            fa = _from_key(ka0[pl.ds(qb, 16)])
            return acc + jnp.abs(fx - fa)
        acc = lax.fori_loop(0, NV, wacc, jnp.zeros((16,), jnp.float32),
                            unroll=8)

        med_off = 2032 + (2032 >> 8)
        medx = _lane15(_from_key(kx0[pl.ds(med_off, 16)]))
        meda = _lane15(_from_key(ka0[pl.ds(med_off, 16)]))
        med_d = medx - meda
        sgn = jnp.sign(med_d)
        inv_n = np.float32(1.0 / N)
        _store_scalar(resm, r, (jnp.sum(sumx) - jnp.sum(suma)) * inv_n * sgn, lane)
        _store_scalar(resw, r, jnp.sum(acc) * inv_n * sgn, lane)
        _store_scalar(resd, r, med_d, lane)
        return c

    lax.fori_loop(0, RPW, row_body, 0)

    pltpu.sync_copy(resm, out_hbm.at[0, pl.ds(base, RPW)])
    pltpu.sync_copy(resw, out_hbm.at[1, pl.ds(base, RPW)])
    pltpu.sync_copy(resd, out_hbm.at[2, pl.ds(base, RPW)])


@functools.lru_cache(maxsize=None)
def _build():
    return pl.kernel(
        _sc_body,
        out_type=jax.ShapeDtypeStruct((3, M), jnp.float32),
        mesh=plsc.VectorSubcoreMesh(core_axis_name="c", subcore_axis_name="s"),
        compiler_params=pltpu.CompilerParams(needs_layout_passes=False),
        scratch_types=[
            pltpu.VMEM((N,), jnp.float32),
            pltpu.VMEM((N,), jnp.float32),
            pltpu.VMEM((N + 16,), jnp.int32),
            pltpu.VMEM((N + 16,), jnp.int32),
            pltpu.VMEM((N + 16,), jnp.int32),
            pltpu.VMEM((N + 16,), jnp.int32),
            pltpu.VMEM((NB * 16,), jnp.int32),
            pltpu.VMEM((NB * 16,), jnp.int32),
            pltpu.VMEM((RPW,), jnp.float32),
            pltpu.VMEM((RPW,), jnp.float32),
            pltpu.VMEM((RPW,), jnp.float32),
        ],
    )


def kernel(x, anchor_features):
    return _build()(x, anchor_features)


# speedup vs baseline: 2.5408x; 1.0388x over previous
import functools

import numpy as np

import jax
import jax.numpy as jnp
from jax import lax
from jax.experimental import pallas as pl
from jax.experimental.pallas import tpu as pltpu
from jax.experimental.pallas import tpu_sc as plsc

M = 2048
N = 4096
NV = N // 16
NB = 256
NC = 2
NS = 16
NW = NC * NS
RPW = M // NW
S = 2
NST = 2 * S
MINI32 = np.int32(-2147483648)


def _to_key(v):
    xi = lax.bitcast_convert_type(v, jnp.int32)
    mask = (xi >> 31) | MINI32
    return xi ^ mask


def _from_key(k):
    mask = ((~k) >> 31) | MINI32
    return lax.bitcast_convert_type(k ^ mask, jnp.float32)


def _digit(k, shift):
    ku = lax.bitcast_convert_type(k, jnp.uint32)
    return ((ku >> shift) & 255).astype(jnp.int32)


def _lane15(v):
    lane = lax.iota(jnp.int32, 16)
    return jnp.sum(jnp.where(lane == 15, v, jnp.zeros((16,), v.dtype)))


def _store_scalar(ref, idx, val, lane):
    idxv = jnp.broadcast_to(idx, (16,)).astype(jnp.int32)
    valv = jnp.broadcast_to(val, (16,))
    plsc.store_scatter(ref, [idxv], valv, mask=lane == 0)


def _sc_body(x_hbm, a_hbm, out_hbm, *scratch):
    inbuf = scratch[0:NST]
    k0 = scratch[NST:2 * NST]
    k1 = scratch[2 * NST:3 * NST]
    cnt = scratch[3 * NST:4 * NST]
    resm, resw, resd = scratch[4 * NST:4 * NST + 3]

    wid = lax.axis_index("s") * NC + lax.axis_index("c")
    base = wid * RPW
    lane = lax.iota(jnp.int32, 16)
    stride_idx = lane * (NV + 1)
    zeros16 = jnp.zeros((16,), jnp.int32)
    ones16 = jnp.ones((16,), jnp.int32)

    def radix_pass(shift, srcs, dsts):
        def zero(i, c):
            for t in range(NST):
                cnt[t][pl.ds(i * 16, 16)] = zeros16
            return c
        lax.fori_loop(0, NB, zero, 0, unroll=8)

        def hist(i, c):
            kv = [plsc.load_gather(srcs[t], [stride_idx + i])
                  for t in range(NST)]
            for t in range(NST):
                d = _digit(kv[t], shift)
                plsc.addupdate_scatter(cnt[t], [d * 16 + lane], ones16)
            return c
        lax.fori_loop(0, NV, hist, 0, unroll=4)

        def prefix(i, carry):
            out = []
            for t in range(NST):
                v = cnt[t][pl.ds(i * 16, 16)]
                cnt[t][pl.ds(i * 16, 16)] = plsc.cumsum(v) - v + carry[t]
                out.append(carry[t] + jnp.sum(v))
            return tuple(out)
        lax.fori_loop(0, NB, prefix, (jnp.int32(0),) * NST, unroll=4)

        def perm(i, c):
            kv = [plsc.load_gather(srcs[t], [stride_idx + i])
                  for t in range(NST)]
            ci = [_digit(kv[t], shift) * 16 + lane for t in range(NST)]
            dest = [plsc.load_gather(cnt[t], [ci[t]]) for t in range(NST)]
            for t in range(NST):
                plsc.store_scatter(cnt[t], [ci[t]], dest[t] + 1)
                plsc.store_scatter(dsts[t], [dest[t] + (dest[t] >> 8)], kv[t])
            return c
        lax.fori_loop(0, NV, perm, 0, unroll=2)

    def row_body(r, c):
        for s in range(S):
            row = base + r * S + s
            pltpu.sync_copy(x_hbm.at[row], inbuf[2 * s])
            pltpu.sync_copy(a_hbm.at[row], inbuf[2 * s + 1])

        def conv(i, carry):
            qb = i * 16 + (i >> 4)
            out = []
            for t in range(NST):
                v = inbuf[t][pl.ds(i * 16, 16)]
                k0[t][pl.ds(qb, 16)] = _to_key(v)
                out.append(carry[t] + v)
            return tuple(out)
        sums = lax.fori_loop(0, NV, conv,
                             (jnp.zeros((16,), jnp.float32),) * NST,
                             unroll=4)

        radix_pass(0, k0, k1)
        radix_pass(8, k1, k0)
        radix_pass(16, k0, k1)
        radix_pass(24, k1, k0)

        def wacc(i, carry):
            qb = i * 16 + (i >> 4)
            out = []
            for s in range(S):
                fx = _from_key(k0[2 * s][pl.ds(qb, 16)])
                fa = _from_key(k0[2 * s + 1][pl.ds(qb, 16)])
                out.append(carry[s] + jnp.abs(fx - fa))
            return tuple(out)
        accs = lax.fori_loop(0, NV, wacc,
                             (jnp.zeros((16,), jnp.float32),) * S,
                             unroll=4)

        med_off = 2032 + (2032 >> 8)
        inv_n = np.float32(1.0 / N)
        for s in range(S):
            medx = _lane15(_from_key(k0[2 * s][pl.ds(med_off, 16)]))
            meda = _lane15(_from_key(k0[2 * s + 1][pl.ds(med_off, 16)]))
            med_d = medx - meda
            sgn = jnp.sign(med_d)
            mean_d = (jnp.sum(sums[2 * s]) - jnp.sum(sums[2 * s + 1])) * inv_n
            idx = r * S + s
            _store_scalar(resm, idx, mean_d * sgn, lane)
            _store_scalar(resw, idx, jnp.sum(accs[s]) * inv_n * sgn, lane)
            _store_scalar(resd, idx, med_d, lane)
        return c

    lax.fori_loop(0, RPW // S, row_body, 0)

    pltpu.sync_copy(resm, out_hbm.at[0, pl.ds(base, RPW)])
    pltpu.sync_copy(resw, out_hbm.at[1, pl.ds(base, RPW)])
    pltpu.sync_copy(resd, out_hbm.at[2, pl.ds(base, RPW)])


@functools.lru_cache(maxsize=None)
def _build():
    scratch = (
        [pltpu.VMEM((N,), jnp.float32) for _ in range(NST)]
        + [pltpu.VMEM((N + 16,), jnp.int32) for _ in range(NST)]
        + [pltpu.VMEM((N + 16,), jnp.int32) for _ in range(NST)]
        + [pltpu.VMEM((NB * 16,), jnp.int32) for _ in range(NST)]
        + [pltpu.VMEM((RPW,), jnp.float32) for _ in range(3)]
    )
    return pl.kernel(
        _sc_body,
        out_type=jax.ShapeDtypeStruct((3, M), jnp.float32),
        mesh=plsc.VectorSubcoreMesh(core_axis_name="c", subcore_axis_name="s"),
        compiler_params=pltpu.CompilerParams(needs_layout_passes=False),
        scratch_types=scratch,
    )


def kernel(x, anchor_features):
    return _build()(x, anchor_features)


# speedup vs baseline: 2.6795x; 1.0546x over previous
import functools

import numpy as np

import jax
import jax.numpy as jnp
from jax import lax
from jax.experimental import pallas as pl
from jax.experimental.pallas import tpu as pltpu
from jax.experimental.pallas import tpu_sc as plsc

M = 2048
N = 4096
NV = N // 16
NB = 256
NC = 2
NS = 16
NW = NC * NS
RPW = M // NW
S = 2
NST = 2 * S
MINI32 = np.int32(-2147483648)


def _to_key(v):
    xi = lax.bitcast_convert_type(v, jnp.int32)
    mask = (xi >> 31) | MINI32
    return xi ^ mask


def _from_key(k):
    mask = ((~k) >> 31) | MINI32
    return lax.bitcast_convert_type(k ^ mask, jnp.float32)


def _digit(k, shift):
    ku = lax.bitcast_convert_type(k, jnp.uint32)
    return ((ku >> shift) & 255).astype(jnp.int32)


def _store_scalar(ref, idx, val, lane):
    idxv = jnp.broadcast_to(idx, (16,)).astype(jnp.int32)
    valv = jnp.broadcast_to(val, (16,))
    plsc.store_scatter(ref, [idxv], valv, mask=lane == 0)


def _sc_body(x_hbm, a_hbm, out_hbm, *scratch):
    inbuf = scratch[0:NST]
    k0 = scratch[NST:2 * NST]
    k1 = scratch[2 * NST:3 * NST]
    cnt0 = scratch[3 * NST:4 * NST]
    cnt1 = scratch[4 * NST:5 * NST]
    resm, resw, resd = scratch[5 * NST:5 * NST + 3]

    wid = lax.axis_index("s") * NC + lax.axis_index("c")
    base = wid * RPW
    lane = lax.iota(jnp.int32, 16)
    stride_idx = lane * (NV + 1)
    zeros16 = jnp.zeros((16,), jnp.int32)
    ones16 = jnp.ones((16,), jnp.int32)

    def radix_pass(shift, next_shift, srcs, dsts, cur, nxt):
        def prefix(i, carry):
            newc = []
            for t in range(NST):
                v = cur[t][pl.ds(i * 16, 16)]
                pcs = plsc.cumsum(v)
                cur[t][pl.ds(i * 16, 16)] = pcs - v + carry[t]
                nxt[t][pl.ds(i * 16, 16)] = zeros16
                newc.append(carry[t] + jnp.sum(v))
            return tuple(newc)
        lax.fori_loop(0, NB, prefix, (jnp.int32(0),) * NST, unroll=4)

        def perm(i, c):
            kv = [plsc.load_gather(srcs[t], [stride_idx + i])
                  for t in range(NST)]
            ci = [_digit(kv[t], shift) * 16 + lane for t in range(NST)]
            dest = [plsc.load_gather(cur[t], [ci[t]]) for t in range(NST)]
            for t in range(NST):
                plsc.store_scatter(cur[t], [ci[t]], dest[t] + 1)
                plsc.store_scatter(dsts[t], [dest[t] + (dest[t] >> 8)], kv[t])
                if next_shift is not None:
                    dn = _digit(kv[t], next_shift)
                    plsc.addupdate_scatter(cnt_for_next[t],
                                           [dn * 16 + lane], ones16)
            return c
        cnt_for_next = nxt
        lax.fori_loop(0, NV, perm, 0, unroll=2)

    def row_body(r, c):
        for s in range(S):
            row = base + r * S + s
            pltpu.sync_copy(x_hbm.at[row], inbuf[2 * s])
            pltpu.sync_copy(a_hbm.at[row], inbuf[2 * s + 1])

        def conv(i, carry):
            qb = i * 16 + (i >> 4)
            out = []
            for t in range(NST):
                v = inbuf[t][pl.ds(i * 16, 16)]
                k = _to_key(v)
                k0[t][pl.ds(qb, 16)] = k
                d = _digit(k, 0)
                plsc.addupdate_scatter(cnt0[t], [d * 16 + lane], ones16)
                out.append(carry[t] + v)
            return tuple(out)
        sums = lax.fori_loop(0, NV, conv,
                             (jnp.zeros((16,), jnp.float32),) * NST,
                             unroll=4)

        radix_pass(0, 8, k0, k1, cnt0, cnt1)
        radix_pass(8, 16, k1, k0, cnt1, cnt0)
        radix_pass(16, 24, k0, k1, cnt0, cnt1)
        radix_pass(24, None, k1, k0, cnt1, cnt0)

        def wacc(i, carry):
            qb = i * 16 + (i >> 4)
            out = []
            for s in range(S):
                fx = _from_key(k0[2 * s][pl.ds(qb, 16)])
                fa = _from_key(k0[2 * s + 1][pl.ds(qb, 16)])
                out.append(carry[s] + jnp.abs(fx - fa))
            for t in range(NST):
                cnt0[t][pl.ds(i * 16, 16)] = zeros16
            return tuple(out)
        accs = lax.fori_loop(0, NV, wacc,
                             (jnp.zeros((16,), jnp.float32),) * S,
                             unroll=4)

        med_off = 2032 + (2032 >> 8)
        inv_n = np.float32(1.0 / N)
        for s in range(S):
            mx = _from_key(k0[2 * s][pl.ds(med_off, 16)])
            ma = _from_key(k0[2 * s + 1][pl.ds(med_off, 16)])
            med_d = jnp.sum(jnp.where(lane == 15, mx - ma, 0.0))
            sgn = jnp.sign(med_d)
            mean_d = (jnp.sum(sums[2 * s]) - jnp.sum(sums[2 * s + 1])) * inv_n
            idx = r * S + s
            _store_scalar(resm, idx, mean_d * sgn, lane)
            _store_scalar(resw, idx, jnp.sum(accs[s]) * inv_n * sgn, lane)
            _store_scalar(resd, idx, med_d, lane)
        return c

    def zero0(i, c):
        for t in range(NST):
            cnt0[t][pl.ds(i * 16, 16)] = zeros16
        return c
    lax.fori_loop(0, NB, zero0, 0, unroll=8)

    lax.fori_loop(0, RPW // S, row_body, 0)

    pltpu.sync_copy(resm, out_hbm.at[0, pl.ds(base, RPW)])
    pltpu.sync_copy(resw, out_hbm.at[1, pl.ds(base, RPW)])
    pltpu.sync_copy(resd, out_hbm.at[2, pl.ds(base, RPW)])


@functools.lru_cache(maxsize=None)
def _build():
    scratch = (
        [pltpu.VMEM((N,), jnp.float32) for _ in range(NST)]
        + [pltpu.VMEM((N + 16,), jnp.int32) for _ in range(NST)]
        + [pltpu.VMEM((N + 16,), jnp.int32) for _ in range(NST)]
        + [pltpu.VMEM((NB * 16,), jnp.int32) for _ in range(NST)]
        + [pltpu.VMEM((NB * 16,), jnp.int32) for _ in range(NST)]
        + [pltpu.VMEM((RPW,), jnp.float32) for _ in range(3)]
    )
    return pl.kernel(
        _sc_body,
        out_type=jax.ShapeDtypeStruct((3, M), jnp.float32),
        mesh=plsc.VectorSubcoreMesh(core_axis_name="c", subcore_axis_name="s"),
        compiler_params=pltpu.CompilerParams(needs_layout_passes=False),
        scratch_types=scratch,
    )


def kernel(x, anchor_features):
    return _build()(x, anchor_features)


# speedup vs baseline: 5.9087x; 2.2051x over previous
import functools

import numpy as np

import jax
import jax.numpy as jnp
from jax import lax
from jax.experimental import pallas as pl
from jax.experimental.pallas import tpu as pltpu
from jax.experimental.pallas import tpu_sc as plsc

M = 2048
N = 4096
NV = N // 16
NB = 256
NC = 2
NS = 16
NW = NC * NS
RPW = M // NW
S = 2
NST = 2 * S
MINI32 = np.int32(-2147483648)


def _to_key(v):
    xi = lax.bitcast_convert_type(v, jnp.int32)
    mask = (xi >> 31) | MINI32
    return xi ^ mask


def _from_key(k):
    mask = ((~k) >> 31) | MINI32
    return lax.bitcast_convert_type(k ^ mask, jnp.float32)


def _digit(k, shift):
    ku = lax.bitcast_convert_type(k, jnp.uint32)
    return ((ku >> shift) & 255).astype(jnp.int32)


def _store_scalar(ref, idx, val, lane):
    idxv = jnp.broadcast_to(idx, (16,)).astype(jnp.int32)
    valv = jnp.broadcast_to(val, (16,))
    plsc.store_scatter(ref, [idxv], valv, mask=lane == 0)


def _sc_body(x_hbm, a_hbm, out_hbm, *scratch):
    inbuf = scratch[0:NST]
    k0 = scratch[NST:2 * NST]
    k1 = scratch[2 * NST:3 * NST]
    cnt0 = scratch[3 * NST:4 * NST]
    cnt1 = scratch[4 * NST:5 * NST]
    resm, resw, resd = scratch[5 * NST:5 * NST + 3]

    wid = lax.axis_index("s") * NC + lax.axis_index("c")
    base = wid * RPW
    lane = lax.iota(jnp.int32, 16)
    stride_idx = lane * (NV + 1)
    zeros16 = jnp.zeros((16,), jnp.int32)
    ones16 = jnp.ones((16,), jnp.int32)

    def radix_pass(shift, next_shift, srcs, dsts, cur, nxt):
        def prefix(i, carry):
            newc = []
            for t in range(NST):
                v = cur[t][pl.ds(i * 16, 16)]
                pcs = plsc.cumsum(v)
                cur[t][pl.ds(i * 16, 16)] = pcs - v + carry[t]
                nxt[t][pl.ds(i * 16, 16)] = zeros16
                newc.append(carry[t] + jnp.sum(v))
            return tuple(newc)
        plsc.parallel_loop(0, NB, carry=(jnp.int32(0),) * NST,
                           unroll=4)(prefix)

        def perm(i, c):
            kv = [plsc.load_gather(srcs[t], [stride_idx + i])
                  for t in range(NST)]
            ci = [_digit(kv[t], shift) * 16 + lane for t in range(NST)]
            dest = [plsc.load_gather(cur[t], [ci[t]]) for t in range(NST)]
            for t in range(NST):
                plsc.store_scatter(cur[t], [ci[t]], dest[t] + 1)
                plsc.store_scatter(dsts[t], [dest[t] + (dest[t] >> 8)], kv[t])
                if next_shift is not None:
                    dn = _digit(kv[t], next_shift)
                    plsc.addupdate_scatter(cnt_for_next[t],
                                           [dn * 16 + lane], ones16)
            return c
        cnt_for_next = nxt
        lax.fori_loop(0, NV, perm, 0, unroll=2)

    def row_body(r, c):
        for s in range(S):
            row = base + r * S + s
            pltpu.sync_copy(x_hbm.at[row], inbuf[2 * s])
            pltpu.sync_copy(a_hbm.at[row], inbuf[2 * s + 1])

        def conv(i, carry):
            qb = i * 16 + (i >> 4)
            out = []
            for t in range(NST):
                v = inbuf[t][pl.ds(i * 16, 16)]
                k = _to_key(v)
                k0[t][pl.ds(qb, 16)] = k
                d = _digit(k, 0)
                plsc.addupdate_scatter(cnt0[t], [d * 16 + lane], ones16)
                out.append(carry[t] + v)
            return tuple(out)
        sums = plsc.parallel_loop(
            0, NV, carry=(jnp.zeros((16,), jnp.float32),) * NST,
            unroll=4)(conv)

        radix_pass(0, 8, k0, k1, cnt0, cnt1)
        radix_pass(8, 16, k1, k0, cnt1, cnt0)
        radix_pass(16, 24, k0, k1, cnt0, cnt1)
        radix_pass(24, None, k1, k0, cnt1, cnt0)

        def wacc(i, carry):
            qb = i * 16 + (i >> 4)
            out = []
            for s in range(S):
                fx = _from_key(k0[2 * s][pl.ds(qb, 16)])
                fa = _from_key(k0[2 * s + 1][pl.ds(qb, 16)])
                out.append(carry[s] + jnp.abs(fx - fa))
            for t in range(NST):
                cnt0[t][pl.ds(i * 16, 16)] = zeros16
            return tuple(out)
        accs = plsc.parallel_loop(
            0, NV, carry=(jnp.zeros((16,), jnp.float32),) * S,
            unroll=4)(wacc)

        med_off = 2032 + (2032 >> 8)
        inv_n = np.float32(1.0 / N)
        for s in range(S):
            mx = _from_key(k0[2 * s][pl.ds(med_off, 16)])
            ma = _from_key(k0[2 * s + 1][pl.ds(med_off, 16)])
            med_d = jnp.sum(jnp.where(lane == 15, mx - ma, 0.0))
            sgn = jnp.sign(med_d)
            mean_d = (jnp.sum(sums[2 * s]) - jnp.sum(sums[2 * s + 1])) * inv_n
            idx = r * S + s
            _store_scalar(resm, idx, mean_d * sgn, lane)
            _store_scalar(resw, idx, jnp.sum(accs[s]) * inv_n * sgn, lane)
            _store_scalar(resd, idx, med_d, lane)
        return c

    def zero0(i, c):
        for t in range(NST):
            cnt0[t][pl.ds(i * 16, 16)] = zeros16
        return c
    lax.fori_loop(0, NB, zero0, 0, unroll=8)

    lax.fori_loop(0, RPW // S, row_body, 0)

    pltpu.sync_copy(resm, out_hbm.at[0, pl.ds(base, RPW)])
    pltpu.sync_copy(resw, out_hbm.at[1, pl.ds(base, RPW)])
    pltpu.sync_copy(resd, out_hbm.at[2, pl.ds(base, RPW)])


@functools.lru_cache(maxsize=None)
def _build():
    scratch = (
        [pltpu.VMEM((N,), jnp.float32) for _ in range(NST)]
        + [pltpu.VMEM((N + 16,), jnp.int32) for _ in range(NST)]
        + [pltpu.VMEM((N + 16,), jnp.int32) for _ in range(NST)]
        + [pltpu.VMEM((NB * 16,), jnp.int32) for _ in range(NST)]
        + [pltpu.VMEM((NB * 16,), jnp.int32) for _ in range(NST)]
        + [pltpu.VMEM((RPW,), jnp.float32) for _ in range(3)]
    )
    return pl.kernel(
        _sc_body,
        out_type=jax.ShapeDtypeStruct((3, M), jnp.float32),
        mesh=plsc.VectorSubcoreMesh(core_axis_name="c", subcore_axis_name="s"),
        compiler_params=pltpu.CompilerParams(needs_layout_passes=False),
        scratch_types=scratch,
    )


def kernel(x, anchor_features):
    return _build()(x, anchor_features)


# speedup vs baseline: 6.4130x; 1.0854x over previous
import functools

import numpy as np

import jax
import jax.numpy as jnp
from jax import lax
from jax.experimental import pallas as pl
from jax.experimental.pallas import tpu as pltpu
from jax.experimental.pallas import tpu_sc as plsc

M = 2048
N = 4096
NV = N // 16
NB = 256
NC = 2
NS = 16
NW = NC * NS
RPW = M // NW
S = 4
NST = 2 * S
MINI32 = np.int32(-2147483648)


def _to_key(xi):
    mask = (xi >> 31) | MINI32
    return xi ^ mask


def _from_key(k):
    mask = ((~k) >> 31) | MINI32
    return lax.bitcast_convert_type(k ^ mask, jnp.float32)


def _digit(k, shift):
    ku = lax.bitcast_convert_type(k, jnp.uint32)
    return ((ku >> shift) & 255).astype(jnp.int32)


def _store_scalar(ref, idx, val, lane):
    idxv = jnp.broadcast_to(idx, (16,)).astype(jnp.int32)
    valv = jnp.broadcast_to(val, (16,))
    plsc.store_scatter(ref, [idxv], valv, mask=lane == 0)


def _sc_body(x_hbm, a_hbm, out_hbm, *scratch):
    k0 = scratch[0:NST]
    k1 = scratch[NST:2 * NST]
    cnt = scratch[2 * NST:3 * NST]
    resm, resw, resd = scratch[3 * NST:3 * NST + 3]

    wid = lax.axis_index("s") * NC + lax.axis_index("c")
    base = wid * RPW
    lane = lax.iota(jnp.int32, 16)
    stride_idx = lane * (NV + 1)
    zeros16 = jnp.zeros((16,), jnp.int32)
    ones16 = jnp.ones((16,), jnp.int32)

    def zero_cnt(unroll=4):
        def zero(i):
            for t in range(NST):
                cnt[t][pl.ds(i * 16, 16)] = zeros16
        plsc.parallel_loop(0, NB, unroll=unroll)(zero)

    def radix_pass(shift, srcs, dsts, first):
        if not first:
            zero_cnt()

            def hist(i):
                for t in range(NST):
                    kv = plsc.load_gather(srcs[t], [stride_idx + i])
                    d = _digit(kv, shift)
                    plsc.addupdate_scatter(cnt[t], [d * 16 + lane], ones16)
            plsc.parallel_loop(0, NV, unroll=2)(hist)

        def prefix(i, carry):
            newc = []
            for t in range(NST):
                v = cnt[t][pl.ds(i * 16, 16)]
                pcs = plsc.cumsum(v)
                cnt[t][pl.ds(i * 16, 16)] = pcs - v + carry[t]
                newc.append(carry[t] + jnp.sum(v))
            return tuple(newc)
        plsc.parallel_loop(0, NB, carry=(jnp.int32(0),) * NST,
                           unroll=2)(prefix)

        def perm(i, c):
            kv = [plsc.load_gather(srcs[t], [stride_idx + i])
                  for t in range(NST)]
            ci = [_digit(kv[t], shift) * 16 + lane for t in range(NST)]
            dest = [plsc.load_gather(cnt[t], [ci[t]]) for t in range(NST)]
            for t in range(NST):
                plsc.store_scatter(cnt[t], [ci[t]], dest[t] + 1)
                plsc.store_scatter(dsts[t], [dest[t] + (dest[t] >> 8)], kv[t])
            return c
        lax.fori_loop(0, NV, perm, 0, unroll=1)

    def row_body(r, c):
        for s in range(S):
            row = base + r * S + s
            pltpu.sync_copy(x_hbm.at[row], k1[2 * s].at[pl.ds(0, N)])
            pltpu.sync_copy(a_hbm.at[row], k1[2 * s + 1].at[pl.ds(0, N)])

        def conv(i):
            qb = i * 16 + (i >> 4)
            for t in range(NST):
                k = _to_key(k1[t][pl.ds(i * 16, 16)])
                k0[t][pl.ds(qb, 16)] = k
                d = _digit(k, 0)
                plsc.addupdate_scatter(cnt[t], [d * 16 + lane], ones16)
        plsc.parallel_loop(0, NV, unroll=2)(conv)

        radix_pass(0, k0, k1, True)
        radix_pass(8, k1, k0, False)
        radix_pass(16, k0, k1, False)
        radix_pass(24, k1, k0, False)

        def wacc(i, carry):
            qb = i * 16 + (i >> 4)
            sums, diffs = carry
            f = [_from_key(k0[t][pl.ds(qb, 16)]) for t in range(NST)]
            nsums = tuple(sums[t] + f[t] for t in range(NST))
            ndiffs = tuple(diffs[s] + jnp.abs(f[2 * s] - f[2 * s + 1])
                           for s in range(S))
            for t in range(NST):
                cnt[t][pl.ds(i * 16, 16)] = zeros16
            return nsums, ndiffs
        zf = jnp.zeros((16,), jnp.float32)
        sums, diffs = plsc.parallel_loop(
            0, NV, carry=((zf,) * NST, (zf,) * S), unroll=2)(wacc)

        med_off = 2032 + (2032 >> 8)
        inv_n = np.float32(1.0 / N)
        for s in range(S):
            mx = _from_key(k0[2 * s][pl.ds(med_off, 16)])
            ma = _from_key(k0[2 * s + 1][pl.ds(med_off, 16)])
            med_d = jnp.sum(jnp.where(lane == 15, mx - ma, 0.0))
            sgn = jnp.sign(med_d)
            mean_d = (jnp.sum(sums[2 * s]) - jnp.sum(sums[2 * s + 1])) * inv_n
            idx = r * S + s
            _store_scalar(resm, idx, mean_d * sgn, lane)
            _store_scalar(resw, idx, jnp.sum(diffs[s]) * inv_n * sgn, lane)
            _store_scalar(resd, idx, med_d, lane)
        return c

    zero_cnt(unroll=8)
    lax.fori_loop(0, RPW // S, row_body, 0)

    pltpu.sync_copy(resm, out_hbm.at[0, pl.ds(base, RPW)])
    pltpu.sync_copy(resw, out_hbm.at[1, pl.ds(base, RPW)])
    pltpu.sync_copy(resd, out_hbm.at[2, pl.ds(base, RPW)])


@functools.lru_cache(maxsize=None)
def _build():
    scratch = (
        [pltpu.VMEM((N + 16,), jnp.int32) for _ in range(NST)]
        + [pltpu.VMEM((N + 16,), jnp.int32) for _ in range(NST)]
        + [pltpu.VMEM((NB * 16,), jnp.int32) for _ in range(NST)]
        + [pltpu.VMEM((RPW,), jnp.float32) for _ in range(3)]
    )
    return pl.kernel(
        _sc_body,
        out_type=jax.ShapeDtypeStruct((3, M), jnp.float32),
        mesh=plsc.VectorSubcoreMesh(core_axis_name="c", subcore_axis_name="s"),
        compiler_params=pltpu.CompilerParams(needs_layout_passes=False),
        scratch_types=scratch,
    )


def kernel(x, anchor_features):
    xi = lax.bitcast_convert_type(x, jnp.int32)
    ai = lax.bitcast_convert_type(anchor_features, jnp.int32)
    return _build()(xi, ai)
